# baseline (speedup 1.0000x reference)
import functools

import jax
import jax.numpy as jnp
from jax import lax
from jax.experimental import pallas as pl
from jax.experimental.pallas import tpu as pltpu
from jax.experimental.pallas import tpu_sc as plsc

_DX = 128
_DY = 26
_DYP = 32
_B = 4
_N = 512
_NEG = -1e30
_ROW = _DY * _DYP
_CH = 32
_CHW = _CH * _ROW
_NCHUNK = _N // _CH


def _et_body(x_ref, w_ref, t_ref, et_ref, last_ref):
    e = jnp.dot(x_ref[0], w_ref[...], preferred_element_type=jnp.float32)
    for y0 in range(_DY):
        et_ref[0, :, y0 * _DYP:(y0 + 1) * _DYP] = (
            e[:, y0:y0 + 1] + t_ref[y0:y0 + 1, :]
        )
    col = lax.broadcasted_iota(jnp.int32, (1, _DYP), 1)
    last_ref[0] = jnp.where(col >= _DY, _NEG, e[_N - 1:_N, :])


def _compute_et(X, Wp, Tp):
    return pl.pallas_call(
        _et_body,
        grid=(_B,),
        in_specs=[
            pl.BlockSpec((1, _N, _DX), lambda b: (b, 0, 0)),
            pl.BlockSpec((_DX, _DYP), lambda b: (0, 0)),
            pl.BlockSpec((_DYP, _DYP), lambda b: (0, 0)),
        ],
        out_specs=[
            pl.BlockSpec((1, _N, _ROW), lambda b: (b, 0, 0)),
            pl.BlockSpec((1, 1, _DYP), lambda b: (b, 0, 0)),
        ],
        out_shape=[
            jax.ShapeDtypeStruct((_B, _N, _ROW), jnp.float32),
            jax.ShapeDtypeStruct((_B, 1, _DYP), jnp.float32),
        ],
    )(X, Wp, Tp)


_sc_mesh = plsc.VectorSubcoreMesh(core_axis_name="c", subcore_axis_name="s")


@functools.partial(
    pl.kernel,
    mesh=_sc_mesh,
    out_type=jax.ShapeDtypeStruct((_B, _N * _DYP), jnp.float32),
    scratch_types=[
        pltpu.VMEM((_CHW,), jnp.float32),
        pltpu.VMEM((_CHW,), jnp.float32),
        pltpu.VMEM((_DYP,), jnp.float32),
        pltpu.VMEM((_N * _DYP,), jnp.int32),
        pltpu.VMEM((_N * _DYP,), jnp.float32),
        pltpu.SemaphoreType.DMA,
        pltpu.SemaphoreType.DMA,
    ],
)
def _sc_decode(et_hbm, last_hbm, out_hbm, et0_v, et1_v, el_v, bp_v, out_v,
               sem0, sem1):
    c = lax.axis_index("c")
    s = lax.axis_index("s")
    w = c * 2 + s

    @pl.when(s < 2)
    def _():
        bufs = (et0_v, et1_v)
        sems = (sem0, sem1)

        pltpu.sync_copy(last_hbm.at[w], el_v)
        pltpu.async_copy(et_hbm.at[w, pl.ds(0, _CHW)], et0_v, sem0)
        pltpu.async_copy(et_hbm.at[w, pl.ds(_CHW, _CHW)], et1_v, sem1)

        def make_fwd_step(buf, base):
            def fwd_step(j, carry):
                l0, l1 = carry
                acc0 = jnp.full((16,), _NEG, jnp.float32)
                acc1 = jnp.full((16,), _NEG, jnp.float32)
                bp0 = jnp.zeros((16,), jnp.int32)
                bp1 = jnp.zeros((16,), jnp.int32)
                for y0 in range(_DY):
                    xl = l0[y0] if y0 < 16 else l1[y0 - 16]
                    et0 = buf[pl.ds(j * _ROW + y0 * _DYP, 16)]
                    et1 = buf[pl.ds(j * _ROW + y0 * _DYP + 16, 16)]
                    c0 = et0 + xl
                    c1 = et1 + xl
                    m0 = c0 > acc0
                    m1 = c1 > acc1
                    acc0 = jnp.where(m0, c0, acc0)
                    acc1 = jnp.where(m1, c1, acc1)
                    bp0 = jnp.where(m0, y0, bp0)
                    bp1 = jnp.where(m1, y0, bp1)
                i = base + j + 1
                bp_v[pl.ds(i * _DYP, 16)] = bp0
                bp_v[pl.ds(i * _DYP + 16, 16)] = bp1
                return acc0, acc1
            return fwd_step

        zeros16 = jnp.zeros((16,), jnp.float32)

        def chunk_body(cc, carry):
            for b in range(2):
                chunk = cc * 2 + b
                base = chunk * _CH
                pltpu.make_async_copy(
                    et_hbm.at[w, pl.ds(base * _ROW, _CHW)], bufs[b], sems[b]
                ).wait()
                nsteps = jnp.minimum(_CH, (_N - 1) - base)
                carry = lax.fori_loop(
                    0, nsteps, make_fwd_step(bufs[b], base), carry)

                @pl.when(chunk + 2 < _NCHUNK)
                def _prefetch():
                    pltpu.async_copy(
                        et_hbm.at[w, pl.ds((chunk + 2) * _CHW, _CHW)],
                        bufs[b], sems[b])
            return carry

        l0, l1 = lax.fori_loop(0, _NCHUNK // 2, chunk_body, (zeros16, zeros16))

        iota0 = lax.iota(jnp.int32, 16)
        iota1 = iota0 + 16

        def _butterfly(v, op):
            for sh in (8, 4, 2, 1):
                v = op(v, v.at[iota0 ^ sh].get(mode="promise_in_bounds"))
            return v

        v0 = el_v[pl.ds(0, 16)] + l0
        v1 = el_v[pl.ds(16, 16)] + l1
        m = jnp.maximum(_butterfly(v0, jnp.maximum), _butterfly(v1, jnp.maximum))
        big = jnp.full((16,), _DYP, jnp.int32)
        a0 = jnp.where(v0 == m, iota0, big)
        a1 = jnp.where((v1 == m) & (iota1 < _DY), iota1, big)
        ans = _butterfly(jnp.minimum(a0, a1), jnp.minimum)

        one = jnp.float32(1.0)
        zero = jnp.float32(0.0)

        def write_row(i, a):
            out_v[pl.ds(i * _DYP, 16)] = jnp.where(iota0 == a, one, zero)
            out_v[pl.ds(i * _DYP + 16, 16)] = jnp.where(iota1 == a, one, zero)

        write_row(_N - 1, ans)

        fifteen = jnp.full((16,), 15, jnp.int32)

        def back_step(j, a):
            i = _N - 2 - j
            b0 = bp_v[pl.ds((i + 1) * _DYP, 16)]
            b1 = bp_v[pl.ds((i + 1) * _DYP + 16, 16)]
            g0 = b0.at[jnp.minimum(a, fifteen)].get(mode="promise_in_bounds")
            g1 = b1.at[jnp.maximum(a - 16, 0)].get(mode="promise_in_bounds")
            nxt = jnp.where(a < 16, g0, g1)
            write_row(i, nxt)
            return nxt

        lax.fori_loop(0, _N - 1, back_step, ans)

        pltpu.sync_copy(out_v, out_hbm.at[w])


def kernel(X, W, T):
    Wp = jnp.pad(W, ((0, 0), (0, _DYP - _DY)))
    Tp = jnp.pad(T, ((0, _DYP - _DY), (0, _DYP - _DY)),
                 constant_values=_NEG)
    et, last = _compute_et(X, Wp, Tp)
    out = _sc_decode(et.reshape(_B, _N * _ROW), last.reshape(_B, _DYP))
    return out.reshape(_B, _N, _DYP)[:, :, :_DY]


# speedup vs baseline: 1.1095x; 1.1095x over previous
import functools

import jax
import jax.numpy as jnp
from jax import lax
from jax.experimental import pallas as pl
from jax.experimental.pallas import tpu as pltpu
from jax.experimental.pallas import tpu_sc as plsc

_DX = 128
_DY = 26
_DYP = 32
_B = 4
_N = 512
_NEG = -1e30
_ROW = _DY * _DYP
_CH = 32
_CHW = _CH * _ROW
_NCHUNK = _N // _CH


def _et_body(x_ref, w3_ref, trep_ref, wp_ref, et_ref, last_ref):
    et_ref[0] = (
        jnp.dot(x_ref[0], w3_ref[...], preferred_element_type=jnp.float32)
        + trep_ref[...]
    )
    e_last = jnp.dot(x_ref[0, _N - 1:_N, :], wp_ref[...],
                     preferred_element_type=jnp.float32)
    col = lax.broadcasted_iota(jnp.int32, (1, _DYP), 1)
    last_ref[0] = jnp.where(col >= _DY, _NEG, e_last)


def _compute_et(X, W3, Trep, Wp):
    return pl.pallas_call(
        _et_body,
        grid=(_B,),
        in_specs=[
            pl.BlockSpec((1, _N, _DX), lambda b: (b, 0, 0)),
            pl.BlockSpec((_DX, _ROW), lambda b: (0, 0)),
            pl.BlockSpec((1, _ROW), lambda b: (0, 0)),
            pl.BlockSpec((_DX, _DYP), lambda b: (0, 0)),
        ],
        out_specs=[
            pl.BlockSpec((1, _N, _ROW), lambda b: (b, 0, 0)),
            pl.BlockSpec((1, 1, _DYP), lambda b: (b, 0, 0)),
        ],
        out_shape=[
            jax.ShapeDtypeStruct((_B, _N, _ROW), jnp.float32),
            jax.ShapeDtypeStruct((_B, 1, _DYP), jnp.float32),
        ],
    )(X, W3, Trep, Wp)


_sc_mesh = plsc.VectorSubcoreMesh(core_axis_name="c", subcore_axis_name="s")


@functools.partial(
    pl.kernel,
    mesh=_sc_mesh,
    out_type=jax.ShapeDtypeStruct((_B, _N * _DYP), jnp.float32),
    scratch_types=[
        pltpu.VMEM((_CHW,), jnp.float32),
        pltpu.VMEM((_CHW,), jnp.float32),
        pltpu.VMEM((_DYP,), jnp.float32),
        pltpu.VMEM((_N * _DYP,), jnp.int32),
        pltpu.VMEM((_N * _DYP,), jnp.float32),
        pltpu.SemaphoreType.DMA,
        pltpu.SemaphoreType.DMA,
    ],
)
def _sc_decode(et_hbm, last_hbm, out_hbm, et0_v, et1_v, el_v, bp_v, out_v,
               sem0, sem1):
    c = lax.axis_index("c")
    s = lax.axis_index("s")
    w = c * 2 + s

    @pl.when(s < 2)
    def _():
        bufs = (et0_v, et1_v)
        sems = (sem0, sem1)

        pltpu.sync_copy(last_hbm.at[w], el_v)
        pltpu.async_copy(et_hbm.at[w, pl.ds(0, _CHW)], et0_v, sem0)
        pltpu.async_copy(et_hbm.at[w, pl.ds(_CHW, _CHW)], et1_v, sem1)

        def make_fwd_step(buf, base):
            def fwd_step(j, carry):
                l0, l1 = carry
                acc0 = jnp.full((16,), _NEG, jnp.float32)
                acc1 = jnp.full((16,), _NEG, jnp.float32)
                bp0 = jnp.zeros((16,), jnp.int32)
                bp1 = jnp.zeros((16,), jnp.int32)
                for y0 in range(_DY):
                    xl = l0[y0] if y0 < 16 else l1[y0 - 16]
                    et0 = buf[pl.ds(j * _ROW + y0 * _DYP, 16)]
                    et1 = buf[pl.ds(j * _ROW + y0 * _DYP + 16, 16)]
                    c0 = et0 + xl
                    c1 = et1 + xl
                    m0 = c0 > acc0
                    m1 = c1 > acc1
                    acc0 = jnp.where(m0, c0, acc0)
                    acc1 = jnp.where(m1, c1, acc1)
                    bp0 = jnp.where(m0, y0, bp0)
                    bp1 = jnp.where(m1, y0, bp1)
                i = base + j + 1
                bp_v[pl.ds(i * _DYP, 16)] = bp0
                bp_v[pl.ds(i * _DYP + 16, 16)] = bp1
                return acc0, acc1
            return fwd_step

        zeros16 = jnp.zeros((16,), jnp.float32)

        def chunk_body(cc, carry):
            for b in range(2):
                chunk = cc * 2 + b
                base = chunk * _CH
                pltpu.make_async_copy(
                    et_hbm.at[w, pl.ds(base * _ROW, _CHW)], bufs[b], sems[b]
                ).wait()
                nsteps = jnp.minimum(_CH, (_N - 1) - base)
                carry = lax.fori_loop(
                    0, nsteps, make_fwd_step(bufs[b], base), carry)

                @pl.when(chunk + 2 < _NCHUNK)
                def _prefetch():
                    pltpu.async_copy(
                        et_hbm.at[w, pl.ds((chunk + 2) * _CHW, _CHW)],
                        bufs[b], sems[b])
            return carry

        l0, l1 = lax.fori_loop(0, _NCHUNK // 2, chunk_body, (zeros16, zeros16))

        iota0 = lax.iota(jnp.int32, 16)
        iota1 = iota0 + 16

        def _butterfly(v, op):
            for sh in (8, 4, 2, 1):
                v = op(v, v.at[iota0 ^ sh].get(mode="promise_in_bounds"))
            return v

        v0 = el_v[pl.ds(0, 16)] + l0
        v1 = el_v[pl.ds(16, 16)] + l1
        m = jnp.maximum(_butterfly(v0, jnp.maximum), _butterfly(v1, jnp.maximum))
        big = jnp.full((16,), _DYP, jnp.int32)
        a0 = jnp.where(v0 == m, iota0, big)
        a1 = jnp.where((v1 == m) & (iota1 < _DY), iota1, big)
        ans = _butterfly(jnp.minimum(a0, a1), jnp.minimum)

        one = jnp.float32(1.0)
        zero = jnp.float32(0.0)

        def write_row(i, a):
            out_v[pl.ds(i * _DYP, 16)] = jnp.where(iota0 == a, one, zero)
            out_v[pl.ds(i * _DYP + 16, 16)] = jnp.where(iota1 == a, one, zero)

        write_row(_N - 1, ans)

        fifteen = jnp.full((16,), 15, jnp.int32)

        def back_step(j, a):
            i = _N - 2 - j
            b0 = bp_v[pl.ds((i + 1) * _DYP, 16)]
            b1 = bp_v[pl.ds((i + 1) * _DYP + 16, 16)]
            g0 = b0.at[jnp.minimum(a, fifteen)].get(mode="promise_in_bounds")
            g1 = b1.at[jnp.maximum(a - 16, 0)].get(mode="promise_in_bounds")
            nxt = jnp.where(a < 16, g0, g1)
            write_row(i, nxt)
            return nxt

        lax.fori_loop(0, _N - 1, back_step, ans)

        pltpu.sync_copy(out_v, out_hbm.at[w])


def kernel(X, W, T):
    Wp = jnp.pad(W, ((0, 0), (0, _DYP - _DY)))
    W3 = jnp.repeat(W, _DYP, axis=1)
    Trep = jnp.pad(T, ((0, 0), (0, _DYP - _DY)),
                   constant_values=_NEG).reshape(1, _ROW)
    et, last = _compute_et(X, W3, Trep, Wp)
    out = _sc_decode(et.reshape(_B, _N * _ROW), last.reshape(_B, _DYP))
    return out.reshape(_B, _N, _DYP)[:, :, :_DY]


# speedup vs baseline: 1.2784x; 1.1522x over previous
import functools

import jax
import jax.numpy as jnp
from jax import lax
from jax.experimental import pallas as pl
from jax.experimental.pallas import tpu as pltpu
from jax.experimental.pallas import tpu_sc as plsc

_DX = 128
_DY = 26
_DYP = 32
_B = 4
_N = 512
_NEG = -1e30


def _emis_body(x_ref, w_ref, t_ref, emis_ref, tp_ref):
    e = jnp.dot(x_ref[0], w_ref[...], preferred_element_type=jnp.float32)
    pad = jnp.full((_N, _DYP - _DY), _NEG, jnp.float32)
    emis_ref[0] = jnp.concatenate([e, pad], axis=1)
    t_colpad = jnp.full((_DY, _DYP - _DY), _NEG, jnp.float32)
    t_rowpad = jnp.full((_DYP - _DY, _DYP), _NEG, jnp.float32)
    tp_ref[...] = jnp.concatenate(
        [jnp.concatenate([t_ref[...], t_colpad], axis=1), t_rowpad], axis=0)


def _compute_emis(X, W, T):
    return pl.pallas_call(
        _emis_body,
        grid=(_B,),
        in_specs=[
            pl.BlockSpec((1, _N, _DX), lambda b: (b, 0, 0)),
            pl.BlockSpec((_DX, _DY), lambda b: (0, 0)),
            pl.BlockSpec((_DY, _DY), lambda b: (0, 0)),
        ],
        out_specs=[
            pl.BlockSpec((1, _N, _DYP), lambda b: (b, 0, 0)),
            pl.BlockSpec((_DYP, _DYP), lambda b: (0, 0)),
        ],
        out_shape=[
            jax.ShapeDtypeStruct((_B, _N, _DYP), jnp.float32),
            jax.ShapeDtypeStruct((_DYP, _DYP), jnp.float32),
        ],
    )(X, W, T)


_sc_mesh = plsc.VectorSubcoreMesh(core_axis_name="c", subcore_axis_name="s")


@functools.partial(
    pl.kernel,
    mesh=_sc_mesh,
    out_type=jax.ShapeDtypeStruct((_B, _N * _DYP), jnp.float32),
    scratch_types=[
        pltpu.VMEM((_N * _DYP,), jnp.float32),
        pltpu.VMEM((_DYP * _DYP,), jnp.float32),
        pltpu.VMEM((_N * _DYP,), jnp.int32),
        pltpu.VMEM((_N * _DYP,), jnp.float32),
    ],
)
def _sc_decode(emis_hbm, t_hbm, out_hbm, emis_v, t_v, bp_v, out_v):
    c = lax.axis_index("c")
    s = lax.axis_index("s")
    w = c * 2 + s

    @pl.when(s < 2)
    def _():
        pltpu.sync_copy(emis_hbm.at[w], emis_v)
        pltpu.sync_copy(t_hbm, t_v)

        def fwd_step(i, carry):
            l0, l1 = carry
            e0 = emis_v[pl.ds((i - 1) * _DYP, 16)]
            e1 = emis_v[pl.ds((i - 1) * _DYP + 16, 16)]
            acc0 = jnp.full((16,), _NEG, jnp.float32)
            acc1 = jnp.full((16,), _NEG, jnp.float32)
            bp0 = jnp.zeros((16,), jnp.int32)
            bp1 = jnp.zeros((16,), jnp.int32)
            for y0 in range(_DY):
                xe = e0[y0] if y0 < 16 else e1[y0 - 16]
                xl = l0[y0] if y0 < 16 else l1[y0 - 16]
                t0 = t_v[pl.ds(y0 * _DYP, 16)]
                t1 = t_v[pl.ds(y0 * _DYP + 16, 16)]
                c0 = (xe + t0) + xl
                c1 = (xe + t1) + xl
                m0 = c0 > acc0
                m1 = c1 > acc1
                acc0 = jnp.where(m0, c0, acc0)
                acc1 = jnp.where(m1, c1, acc1)
                bp0 = jnp.where(m0, y0, bp0)
                bp1 = jnp.where(m1, y0, bp1)
            bp_v[pl.ds(i * _DYP, 16)] = bp0
            bp_v[pl.ds(i * _DYP + 16, 16)] = bp1
            return acc0, acc1

        zeros16 = jnp.zeros((16,), jnp.float32)
        l0, l1 = lax.fori_loop(1, _N, fwd_step, (zeros16, zeros16))

        iota0 = lax.iota(jnp.int32, 16)
        iota1 = iota0 + 16

        def _butterfly(v, op):
            for sh in (8, 4, 2, 1):
                v = op(v, v.at[iota0 ^ sh].get(mode="promise_in_bounds"))
            return v

        v0 = emis_v[pl.ds((_N - 1) * _DYP, 16)] + l0
        v1 = emis_v[pl.ds((_N - 1) * _DYP + 16, 16)] + l1
        m = jnp.maximum(_butterfly(v0, jnp.maximum), _butterfly(v1, jnp.maximum))
        big = jnp.full((16,), _DYP, jnp.int32)
        a0 = jnp.where(v0 == m, iota0, big)
        a1 = jnp.where((v1 == m) & (iota1 < _DY), iota1, big)
        ans = _butterfly(jnp.minimum(a0, a1), jnp.minimum)

        one = jnp.float32(1.0)
        zero = jnp.float32(0.0)

        def write_row(i, a):
            out_v[pl.ds(i * _DYP, 16)] = jnp.where(iota0 == a, one, zero)
            out_v[pl.ds(i * _DYP + 16, 16)] = jnp.where(iota1 == a, one, zero)

        write_row(_N - 1, ans)

        fifteen = jnp.full((16,), 15, jnp.int32)

        def back_step(j, a):
            i = _N - 2 - j
            b0 = bp_v[pl.ds((i + 1) * _DYP, 16)]
            b1 = bp_v[pl.ds((i + 1) * _DYP + 16, 16)]
            g0 = b0.at[jnp.minimum(a, fifteen)].get(mode="promise_in_bounds")
            g1 = b1.at[jnp.maximum(a - 16, 0)].get(mode="promise_in_bounds")
            nxt = jnp.where(a < 16, g0, g1)
            write_row(i, nxt)
            return nxt

        lax.fori_loop(0, _N - 1, back_step, ans)

        pltpu.sync_copy(out_v, out_hbm.at[w])


def kernel(X, W, T):
    emis, tp = _compute_emis(X, W, T)
    out = _sc_decode(emis.reshape(_B, _N * _DYP), tp.reshape(_DYP * _DYP))
    return out.reshape(_B, _N, _DYP)[:, :, :_DY]
